# baseline (speedup 1.0000x reference)
import functools

import jax
import jax.numpy as jnp
from jax import lax
from jax.experimental import pallas as pl
from jax.experimental.pallas import tpu as pltpu
from jax.experimental.pallas import tpu_sc as plsc

NB_CLASSES = 1000
EMB = 50
WPR = EMB // 2
TS = WPR + 2
V = 20
LANES = 16
NC = 2
NS = 16
NW = NC * NS
RING = 4
CHUNK = 8
TAGW = 1024


def _sc_pool_kernel(n_rows: int):
    rpw = n_rows // NW
    groups = rpw // LANES
    gw = LANES * EMB
    assert rpw == groups * LANES and groups >= RING
    mesh = plsc.VectorSubcoreMesh(
        core_axis_name="c", subcore_axis_name="s",
        num_cores=NC, num_subcores=NS,
    )

    @functools.partial(
        pl.kernel,
        out_type=jax.ShapeDtypeStruct((n_rows * EMB,), jnp.float32),
        mesh=mesh,
        scratch_types=[
            pltpu.VMEM((NB_CLASSES * TS + 2 * LANES,), jnp.int32),
            pltpu.VMEM((V * rpw,), jnp.int32),
            pltpu.VMEM((V * LANES,), jnp.int32),
            pltpu.VMEM((LANES * TAGW,), jnp.int32),
            pltpu.VMEM((RING * gw,), jnp.float32),
            pltpu.SemaphoreType.DMA,
        ],
        compiler_params=pltpu.CompilerParams(needs_layout_passes=False),
    )
    def kern(x_hbm, tab_hbm, out_hbm, tab_v, x_v, a_v, tag_v, stage, sem):
        wid = lax.axis_index("s") * NC + lax.axis_index("c")
        base = wid * rpw
        pltpu.sync_copy(tab_hbm, tab_v.at[pl.ds(0, NB_CLASSES * TS)])
        izero = jnp.zeros((LANES,), jnp.int32)
        for k in range(2):
            tab_v[pl.ds(NB_CLASSES * TS + k * LANES, LANES)] = izero
        pltpu.sync_copy(x_hbm.at[pl.ds(base * V, rpw * V)], x_v)

        one = jnp.full((LANES,), 1.0, jnp.float32)
        himask = jnp.full((LANES,), -65536, jnp.int32)
        lane_tag = lax.iota(jnp.int32, LANES) * TAGW
        lane_out = lax.iota(jnp.int32, LANES) * EMB
        lane_x = lax.iota(jnp.int32, LANES) * V

        def group_body(g, _):
            r0 = g * LANES
            xrow = lane_x + r0 * V
            xs = [plsc.load_gather(x_v, [xrow + v]) for v in range(V)]
            tags = [xs[v] + lane_tag for v in range(V)]
            for v in range(V):
                plsc.store_scatter(tag_v, [tags[v]],
                                   jnp.full((LANES,), v, jnp.int32))
            cnt = jnp.zeros((LANES,), jnp.float32)
            for v in range(V):
                got = plsc.load_gather(tag_v, [tags[v]])
                m = got == v
                cnt = cnt + jnp.where(m, 1.0, 0.0).astype(jnp.float32)
                a_v[pl.ds(v * LANES, LANES)] = (
                    jnp.where(m, xs[v], NB_CLASSES) * TS)
            inv = one / cnt

            slot = g % RING
            sb = slot * gw
            @pl.when(g >= RING)
            def _():
                pltpu.make_async_copy(
                    stage.at[pl.ds(sb, gw)],
                    out_hbm.at[pl.ds(base * EMB, gw)],
                    sem,
                ).wait()

            for c0 in range(0, EMB, CHUNK):
                nd = min(CHUNK, EMB - c0)
                nk = (nd + 1) // 2
                accs = [jnp.zeros((LANES,), jnp.float32)] * nd
                for v in range(V):
                    av = a_v[pl.ds(v * LANES, LANES)] + (c0 // 2)
                    for k in range(nk):
                        w32 = plsc.load_gather(tab_v, [av + k])
                        lo = plsc.bitcast(lax.shift_left(w32, 16), jnp.float32)
                        accs[2 * k] = accs[2 * k] + lo
                        if 2 * k + 1 < nd:
                            hi = plsc.bitcast(w32 & himask, jnp.float32)
                            accs[2 * k + 1] = accs[2 * k + 1] + hi
                for j in range(nd):
                    plsc.store_scatter(stage, [lane_out + (sb + c0 + j)],
                                       accs[j] * inv)

            pltpu.make_async_copy(
                stage.at[pl.ds(sb, gw)],
                out_hbm.at[pl.ds((base + r0) * EMB, gw)],
                sem,
            ).start()
            return 0

        lax.fori_loop(0, groups, group_body, 0)
        for _ in range(RING):
            pltpu.make_async_copy(
                stage.at[pl.ds(0, gw)],
                out_hbm.at[pl.ds(base * EMB, gw)],
                sem,
            ).wait()

    return kern


def kernel(x, table):
    b, l, v = x.shape
    n = b * l
    x_t = x.reshape(-1).astype(jnp.int32)
    bits = lax.bitcast_convert_type(table.astype(jnp.bfloat16), jnp.uint16)
    words = (bits[:, 0::2].astype(jnp.uint32)
             | (bits[:, 1::2].astype(jnp.uint32) << 16))
    words = jnp.pad(words, ((0, 0), (0, TS - WPR)))
    tab_p = lax.bitcast_convert_type(words, jnp.int32).reshape(-1)
    out = _sc_pool_kernel(n)(x_t, tab_p)
    return out.reshape(b, l, EMB)


# speedup vs baseline: 1.0479x; 1.0479x over previous
import functools

import jax
import jax.numpy as jnp
from jax import lax
from jax.experimental import pallas as pl
from jax.experimental.pallas import tpu as pltpu
from jax.experimental.pallas import tpu_sc as plsc

NB_CLASSES = 1000
EMB = 50
WPR = EMB // 2
TS = WPR + 2
V = 20
LANES = 16
NC = 2
NS = 16
NW = NC * NS
RING = 4
CHUNK = 10
TAGW = 1024


def _sc_pool_kernel(n_rows: int):
    rpw = n_rows // NW
    groups = rpw // LANES
    gw = LANES * EMB
    assert rpw == groups * LANES and groups >= RING
    mesh = plsc.VectorSubcoreMesh(
        core_axis_name="c", subcore_axis_name="s",
        num_cores=NC, num_subcores=NS,
    )

    @functools.partial(
        pl.kernel,
        out_type=jax.ShapeDtypeStruct((n_rows * EMB,), jnp.float32),
        mesh=mesh,
        scratch_types=[
            pltpu.VMEM((NB_CLASSES * TS + 2 * LANES,), jnp.int32),
            pltpu.VMEM((V * rpw,), jnp.int32),
            pltpu.VMEM((V * LANES,), jnp.int32),
            pltpu.VMEM((LANES * TAGW,), jnp.int32),
            pltpu.VMEM((RING * gw,), jnp.float32),
            pltpu.SemaphoreType.DMA,
        ],
        compiler_params=pltpu.CompilerParams(needs_layout_passes=False),
    )
    def kern(x_hbm, tab_hbm, out_hbm, tab_v, x_v, a_v, tag_v, stage, sem):
        wid = lax.axis_index("s") * NC + lax.axis_index("c")
        base = wid * rpw
        pltpu.sync_copy(tab_hbm, tab_v.at[pl.ds(0, NB_CLASSES * TS)])
        izero = jnp.zeros((LANES,), jnp.int32)
        for k in range(2):
            tab_v[pl.ds(NB_CLASSES * TS + k * LANES, LANES)] = izero
        for v in range(V):
            pltpu.sync_copy(x_hbm.at[pl.ds(v * n_rows + base, rpw)],
                            x_v.at[pl.ds(v * rpw, rpw)])

        one = jnp.full((LANES,), 1.0, jnp.float32)
        himask = jnp.full((LANES,), -65536, jnp.int32)
        lane_tag = lax.iota(jnp.int32, LANES) * TAGW
        lane_out = lax.iota(jnp.int32, LANES) * EMB
        lane_x = lax.iota(jnp.int32, LANES) * V

        def group_body(g, _):
            r0 = g * LANES
            xs = [x_v[pl.ds(v * rpw + r0, LANES)] for v in range(V)]
            tags = [xs[v] + lane_tag for v in range(V)]
            for v in range(V):
                plsc.store_scatter(tag_v, [tags[v]],
                                   jnp.full((LANES,), v, jnp.int32))
            cnt = jnp.zeros((LANES,), jnp.float32)
            for v in range(V):
                got = plsc.load_gather(tag_v, [tags[v]])
                m = got == v
                cnt = cnt + jnp.where(m, 1.0, 0.0).astype(jnp.float32)
                a_v[pl.ds(v * LANES, LANES)] = (
                    jnp.where(m, xs[v], NB_CLASSES) * TS)
            inv = one / cnt

            slot = g % RING
            sb = slot * gw
            @pl.when(g >= RING)
            def _():
                pltpu.make_async_copy(
                    stage.at[pl.ds(sb, gw)],
                    out_hbm.at[pl.ds(base * EMB, gw)],
                    sem,
                ).wait()

            for c0 in range(0, EMB, CHUNK):
                nd = min(CHUNK, EMB - c0)
                nk = (nd + 1) // 2
                accs = [jnp.zeros((LANES,), jnp.float32)] * nd
                for v in range(V):
                    av = a_v[pl.ds(v * LANES, LANES)] + (c0 // 2)
                    for k in range(nk):
                        w32 = plsc.load_gather(tab_v, [av + k])
                        lo = plsc.bitcast(lax.shift_left(w32, 16), jnp.float32)
                        accs[2 * k] = accs[2 * k] + lo
                        if 2 * k + 1 < nd:
                            hi = plsc.bitcast(w32 & himask, jnp.float32)
                            accs[2 * k + 1] = accs[2 * k + 1] + hi
                for j in range(nd):
                    plsc.store_scatter(stage, [lane_out + (sb + c0 + j)],
                                       accs[j] * inv)

            pltpu.make_async_copy(
                stage.at[pl.ds(sb, gw)],
                out_hbm.at[pl.ds((base + r0) * EMB, gw)],
                sem,
            ).start()
            return 0

        lax.fori_loop(0, groups, group_body, 0)
        for _ in range(RING):
            pltpu.make_async_copy(
                stage.at[pl.ds(0, gw)],
                out_hbm.at[pl.ds(base * EMB, gw)],
                sem,
            ).wait()

    return kern


def kernel(x, table):
    b, l, v = x.shape
    n = b * l
    x_t = x.reshape(n, v).astype(jnp.int32).T.reshape(-1)
    bits = lax.bitcast_convert_type(table.astype(jnp.bfloat16), jnp.uint16)
    words = (bits[:, 0::2].astype(jnp.uint32)
             | (bits[:, 1::2].astype(jnp.uint32) << 16))
    words = jnp.pad(words, ((0, 0), (0, TS - WPR)))
    tab_p = lax.bitcast_convert_type(words, jnp.int32).reshape(-1)
    out = _sc_pool_kernel(n)(x_t, tab_p)
    return out.reshape(b, l, EMB)
